# baseline (speedup 1.0000x reference)
import functools

import jax
import jax.numpy as jnp
from jax import lax
from jax.experimental import pallas as pl
from jax.experimental.pallas import tpu as pltpu
from jax.experimental.pallas import tpu_sc as plsc

_B, _C, _H, _W = 8, 768, 32, 32
_HW = _H * _W
_SCALES = (
    (4, 29, 6, 0),
    (6, 27, 5, 841),
    (8, 25, 4, 1570),
)
_NUM_PROPOSALS = 15
_NEG_INF = float("-inf")


def _lane_reduce(vec, op):
    vals = [vec[i] for i in range(16)]
    while len(vals) > 1:
        vals = [op(vals[i], vals[i + 1]) for i in range(0, len(vals), 2)]
    return vals[0]


def _pool_2d(fm, r):
    def horiz(a_w, w, b_v):
        return a_w + jnp.roll(b_v, -w, axis=1)

    def vert(a_w, w, b_v):
        return a_w + jnp.roll(b_v, -w, axis=0)

    h2 = horiz(fm, 1, fm)
    h4 = horiz(h2, 2, h2)
    if r == 4:
        hs = h4
    elif r == 6:
        hs = horiz(h4, 4, h2)
    else:
        hs = horiz(h4, 4, h4)
    v2 = vert(hs, 1, hs)
    v4 = vert(v2, 2, v2)
    if r == 4:
        ps = v4
    elif r == 6:
        ps = vert(v4, 4, v2)
    else:
        ps = vert(v4, 4, v4)
    return ps * (1.0 / float(r * r))


def _tc_body(x_ref, out_ref):
    fm = jnp.sum(x_ref[0], axis=0)
    for j, (r, _, _, _) in enumerate(_SCALES):
        out_ref[0, j] = _pool_2d(fm, r)


@jax.jit
def _tc_scores(x):
    return pl.pallas_call(
        _tc_body,
        grid=(_B,),
        in_specs=[pl.BlockSpec((1, _C, _H, _W), lambda b: (b, 0, 0, 0))],
        out_specs=pl.BlockSpec((1, 3, _H, _W), lambda b: (b, 0, 0, 0)),
        out_shape=jax.ShapeDtypeStruct((_B, 3, _H, _W), jnp.float32),
    )(x)


def _sc_nms_scale(r, side, nsel, base, b, s_ref, mask_ref, idxv_ref, scrv_ref):
    iota = lax.broadcasted_iota(jnp.int32, (16,), 0)

    def init_row(i, _):
        for h in range(2):
            mask_ref[i, pl.ds(h * 16, 16)] = jnp.where(
                h * 16 + iota < side, 0.0, _NEG_INF)
        return 0

    lax.fori_loop(0, side, init_row, 0)

    out_idx = jnp.zeros((16,), jnp.int32)
    out_scr = jnp.zeros((16,), jnp.float32)
    for k in range(nsel):
        def max_row(i, vmax):
            for h in range(2):
                d = pl.ds(h * 16, 16)
                vmax = jnp.maximum(vmax, s_ref[i, d] + mask_ref[i, d])
            return vmax

        m = _lane_reduce(
            lax.fori_loop(0, side, max_row,
                          jnp.full((16,), _NEG_INF, jnp.float32)),
            jnp.maximum)

        def arg_row(i, vmin):
            for h in range(2):
                d = pl.ds(h * 16, 16)
                p = i * 32 + h * 16 + iota
                cand = jnp.where(s_ref[i, d] + mask_ref[i, d] == m, p,
                                 jnp.int32(2**30))
                vmin = jnp.minimum(vmin, cand)
            return vmin

        p32 = _lane_reduce(
            lax.fori_loop(0, side, arg_row,
                          jnp.full((16,), 2**30, jnp.int32)),
            jnp.minimum)
        i0 = lax.shift_right_logical(p32, 5)
        j0 = lax.bitwise_and(p32, 31)

        def supp_row(ii, _):
            u = r - jnp.abs(ii - i0)
            for h in range(2):
                pj = h * 16 + iota
                v = jnp.maximum(0, r - jnp.abs(pj - j0))
                cond = 5 * u * v > 2 * r * r
                d = pl.ds(h * 16, 16)
                mask_ref[ii, d] = jnp.where(cond, _NEG_INF, mask_ref[ii, d])
            return 0

        lax.fori_loop(jnp.maximum(0, i0 - (r - 1)),
                      jnp.minimum(side, i0 + r), supp_row, 0)

        gidx = i0 * side + j0 + base
        out_idx = jnp.where(iota == k, gidx, out_idx)
        out_scr = jnp.where(iota == k, m, out_scr)

    idxv_ref[...] = out_idx
    scrv_ref[...] = out_scr


def _sc_nms_kernel():
    info = plsc.get_sparse_core_info()
    nc = info.num_cores

    @functools.partial(
        pl.kernel,
        mesh=plsc.VectorSubcoreMesh(core_axis_name="c", subcore_axis_name="s"),
        out_type=(
            jax.ShapeDtypeStruct((_B, 3, 16), jnp.int32),
            jax.ShapeDtypeStruct((_B, 3, 16), jnp.float32),
        ),
        scratch_types=[
            pltpu.VMEM((_H, _W), jnp.float32),
            pltpu.VMEM((_H, _W), jnp.float32),
            pltpu.VMEM((16,), jnp.int32),
            pltpu.VMEM((16,), jnp.float32),
        ],
    )
    def nms(scores_hbm, idx_hbm, scr_hbm, s_v, mask_v, idxv, scrv):
        wid = lax.axis_index("s") * nc + lax.axis_index("c")
        b = wid % _B
        j = wid // _B

        @pl.when(wid < _B * 3)
        def _():
            pltpu.sync_copy(scores_hbm.at[b, j], s_v)
            for jj, (r, side, nsel, base) in enumerate(_SCALES):
                @pl.when(j == jj)
                def _():
                    _sc_nms_scale(r, side, nsel, base, b,
                                  s_v, mask_v, idxv, scrv)
            pltpu.sync_copy(idxv, idx_hbm.at[b, j])
            pltpu.sync_copy(scrv, scr_hbm.at[b, j])

    return nms


@jax.jit
def _run(input_tensor):
    packed = _tc_scores(input_tensor)
    idx_p, scr_p = _sc_nms_kernel()(packed)
    maps = packed
    window_scores = jnp.concatenate(
        [maps[:, jj, :side, :side].reshape(_B, side * side)
         for jj, (_, side, _, _) in enumerate(_SCALES)], axis=1)
    proposal_indices = jnp.concatenate(
        [idx_p[:, jj, :nsel] for jj, (_, _, nsel, _) in enumerate(_SCALES)],
        axis=1)
    proposal_scores = jnp.concatenate(
        [scr_p[:, jj, :nsel] for jj, (_, _, nsel, _) in enumerate(_SCALES)],
        axis=1)
    return proposal_indices, proposal_scores, window_scores


def kernel(input_tensor, coordinates_cat, num_proposals, pooling_ratios,
           window_nums_sum, N_list, iou_thresholds):
    return _run(input_tensor)


# speedup vs baseline: 2.3598x; 2.3598x over previous
import functools

import jax
import jax.numpy as jnp
from jax import lax
from jax.experimental import pallas as pl
from jax.experimental.pallas import tpu as pltpu
from jax.experimental.pallas import tpu_sc as plsc

_B, _C, _H, _W = 8, 768, 32, 32
_HW = _H * _W
_SCALES = (
    (4, 29, 6, 0),
    (6, 27, 5, 841),
    (8, 25, 4, 1570),
)
_TOTAL = 2195
_NUM_PROPOSALS = 15
_NEG_INF = float("-inf")


def _lane_reduce(vec, op):
    vals = [vec[i] for i in range(16)]
    while len(vals) > 1:
        vals = [op(vals[i], vals[i + 1]) for i in range(0, len(vals), 2)]
    return vals[0]


def _pool_1d(fm, r):
    def widen(a, w, b):
        return a + jnp.roll(b, -w, axis=1)

    h2 = widen(fm, 1, fm)
    h4 = widen(h2, 2, h2)
    if r == 4:
        hs = h4
    elif r == 6:
        hs = widen(h4, 4, h2)
    else:
        hs = widen(h4, 4, h4)
    v2 = widen(hs, 32, hs)
    v4 = widen(v2, 64, v2)
    if r == 4:
        ps = v4
    elif r == 6:
        ps = widen(v4, 128, v2)
    else:
        ps = widen(v4, 128, v4)
    return ps * (1.0 / float(r * r))


def _tc_body(x_ref, out_ref):
    fm = jnp.sum(x_ref[0], axis=0, keepdims=True)
    for j, (r, _, _, _) in enumerate(_SCALES):
        out_ref[0, j] = _pool_1d(fm, r)[0]


@jax.jit
def _tc_scores(x):
    return pl.pallas_call(
        _tc_body,
        grid=(_B,),
        in_specs=[pl.BlockSpec((1, _C, _HW), lambda b: (b, 0, 0))],
        out_specs=pl.BlockSpec((1, 3, _HW), lambda b: (b, 0, 0)),
        out_shape=jax.ShapeDtypeStruct((_B, 3, _HW), jnp.float32),
    )(x)


def _asm_body(maps_ref, idx_ref, scr_ref, ws_ref, pidx_ref, pscr_ref):
    parts = []
    for j, (_, side, _, _) in enumerate(_SCALES):
        for i in range(side):
            parts.append(maps_ref[:, j, 32 * i:32 * i + side])
    ws_ref[...] = jnp.concatenate(parts, axis=1)
    ip, sp = [], []
    for j, (_, _, nsel, _) in enumerate(_SCALES):
        ip.append(idx_ref[:, 16 * j:16 * j + nsel])
        sp.append(scr_ref[:, 16 * j:16 * j + nsel])
    pidx_ref[...] = jnp.concatenate(ip, axis=1)
    pscr_ref[...] = jnp.concatenate(sp, axis=1)


def _assemble(maps, idx48, scr48):
    return pl.pallas_call(
        _asm_body,
        out_shape=(
            jax.ShapeDtypeStruct((_B, _TOTAL), jnp.float32),
            jax.ShapeDtypeStruct((_B, _NUM_PROPOSALS), jnp.int32),
            jax.ShapeDtypeStruct((_B, _NUM_PROPOSALS), jnp.float32),
        ),
    )(maps, idx48, scr48)


def _sc_nms_scale(r, side, nsel, base, s_ref, mask_ref, idxv_ref, scrv_ref):
    iota = lax.broadcasted_iota(jnp.int32, (16,), 0)

    def init_row(i, _):
        for h in range(2):
            mask_ref[pl.ds(i * 32 + h * 16, 16)] = jnp.where(
                h * 16 + iota < side, 0.0, _NEG_INF)
        return 0

    lax.fori_loop(0, side, init_row, 0)

    out_idx = jnp.zeros((16,), jnp.int32)
    out_scr = jnp.zeros((16,), jnp.float32)
    for k in range(nsel):
        def max_row(i, vmax):
            for h in range(2):
                d = pl.ds(i * 32 + h * 16, 16)
                vmax = jnp.maximum(vmax, s_ref[d] + mask_ref[d])
            return vmax

        m = _lane_reduce(
            lax.fori_loop(0, side, max_row,
                          jnp.full((16,), _NEG_INF, jnp.float32)),
            jnp.maximum)

        def arg_row(i, vmin):
            for h in range(2):
                d = pl.ds(i * 32 + h * 16, 16)
                p = i * 32 + h * 16 + iota
                cand = jnp.where(s_ref[d] + mask_ref[d] == m, p,
                                 jnp.int32(2**30))
                vmin = jnp.minimum(vmin, cand)
            return vmin

        p32 = _lane_reduce(
            lax.fori_loop(0, side, arg_row,
                          jnp.full((16,), 2**30, jnp.int32)),
            jnp.minimum)
        i0 = lax.shift_right_logical(p32, 5)
        j0 = lax.bitwise_and(p32, 31)

        def supp_row(ii, _):
            u = r - jnp.abs(ii - i0)
            for h in range(2):
                pj = h * 16 + iota
                v = jnp.maximum(0, r - jnp.abs(pj - j0))
                cond = 5 * u * v > 2 * r * r
                d = pl.ds(ii * 32 + h * 16, 16)
                mask_ref[d] = jnp.where(cond, _NEG_INF, mask_ref[d])
            return 0

        lax.fori_loop(jnp.maximum(0, i0 - (r - 1)),
                      jnp.minimum(side, i0 + r), supp_row, 0)

        gidx = i0 * side + j0 + base
        out_idx = jnp.where(iota == k, gidx, out_idx)
        out_scr = jnp.where(iota == k, m, out_scr)

    idxv_ref[...] = out_idx
    scrv_ref[...] = out_scr


def _sc_nms_kernel():
    info = plsc.get_sparse_core_info()
    nc = info.num_cores

    @functools.partial(
        pl.kernel,
        mesh=plsc.VectorSubcoreMesh(core_axis_name="c", subcore_axis_name="s"),
        out_type=(
            jax.ShapeDtypeStruct((_B, 48), jnp.int32),
            jax.ShapeDtypeStruct((_B, 48), jnp.float32),
        ),
        scratch_types=[
            pltpu.VMEM((_HW,), jnp.float32),
            pltpu.VMEM((_HW,), jnp.float32),
            pltpu.VMEM((16,), jnp.int32),
            pltpu.VMEM((16,), jnp.float32),
        ],
    )
    def nms(scores_hbm, idx_hbm, scr_hbm, s_v, mask_v, idxv, scrv):
        wid = lax.axis_index("s") * nc + lax.axis_index("c")
        b = wid % _B
        j = wid // _B

        @pl.when(wid < _B * 3)
        def _():
            pltpu.sync_copy(scores_hbm.at[b, j], s_v)
            for jj, (r, side, nsel, base) in enumerate(_SCALES):
                @pl.when(j == jj)
                def _():
                    _sc_nms_scale(r, side, nsel, base, s_v, mask_v, idxv, scrv)
            pltpu.sync_copy(idxv, idx_hbm.at[b, pl.ds(16 * j, 16)])
            pltpu.sync_copy(scrv, scr_hbm.at[b, pl.ds(16 * j, 16)])

    return nms


@jax.jit
def _run(input_tensor):
    packed = _tc_scores(input_tensor.reshape(_B, _C, _HW))
    idx48, scr48 = _sc_nms_kernel()(packed)
    window_scores, proposal_indices, proposal_scores = _assemble(
        packed, idx48, scr48)
    return proposal_indices, proposal_scores, window_scores


def kernel(input_tensor, coordinates_cat, num_proposals, pooling_ratios,
           window_nums_sum, N_list, iou_thresholds):
    return _run(input_tensor)


# speedup vs baseline: 2.3691x; 1.0039x over previous
import functools

import jax
import jax.numpy as jnp
from jax import lax
from jax.experimental import pallas as pl
from jax.experimental.pallas import tpu as pltpu
from jax.experimental.pallas import tpu_sc as plsc

_B, _C, _H, _W = 8, 768, 32, 32
_HW = _H * _W
_SCALES = (
    (4, 29, 6, 0),
    (6, 27, 5, 841),
    (8, 25, 4, 1570),
)
_TOTAL = 2195
_NUM_PROPOSALS = 15
_NEG_INF = float("-inf")


def _lane_reduce(vec, op):
    vals = [vec[i] for i in range(16)]
    while len(vals) > 1:
        vals = [op(vals[i], vals[i + 1]) for i in range(0, len(vals), 2)]
    return vals[0]


def _pool_1d(fm, r):
    def widen(a, w, b):
        return a + jnp.roll(b, -w, axis=1)

    h2 = widen(fm, 1, fm)
    h4 = widen(h2, 2, h2)
    if r == 4:
        hs = h4
    elif r == 6:
        hs = widen(h4, 4, h2)
    else:
        hs = widen(h4, 4, h4)
    v2 = widen(hs, 32, hs)
    v4 = widen(v2, 64, v2)
    if r == 4:
        ps = v4
    elif r == 6:
        ps = widen(v4, 128, v2)
    else:
        ps = widen(v4, 128, v4)
    return ps * (1.0 / float(r * r))


def _tc_body(x_ref, out_ref):
    fm = jnp.sum(x_ref[0], axis=0, keepdims=True)
    for j, (r, _, _, _) in enumerate(_SCALES):
        out_ref[0, j] = _pool_1d(fm, r)[0]


@jax.jit
def _tc_scores(x):
    return pl.pallas_call(
        _tc_body,
        grid=(_B,),
        in_specs=[pl.BlockSpec((1, _C, _HW), lambda b: (b, 0, 0))],
        out_specs=pl.BlockSpec((1, 3, _HW), lambda b: (b, 0, 0)),
        out_shape=jax.ShapeDtypeStruct((_B, 3, _HW), jnp.float32),
        compiler_params=pltpu.CompilerParams(
            dimension_semantics=("parallel",)),
    )(x)


def _asm_body(maps_ref, idx_ref, scr_ref, ws_ref, pidx_ref, pscr_ref):
    parts = []
    for j, (_, side, _, _) in enumerate(_SCALES):
        for i in range(side):
            parts.append(maps_ref[:, j, 32 * i:32 * i + side])
    ws_ref[...] = jnp.concatenate(parts, axis=1)
    ip, sp = [], []
    for j, (_, _, nsel, _) in enumerate(_SCALES):
        ip.append(idx_ref[:, 16 * j:16 * j + nsel])
        sp.append(scr_ref[:, 16 * j:16 * j + nsel])
    pidx_ref[...] = jnp.concatenate(ip, axis=1)
    pscr_ref[...] = jnp.concatenate(sp, axis=1)


def _assemble(maps, idx48, scr48):
    return pl.pallas_call(
        _asm_body,
        out_shape=(
            jax.ShapeDtypeStruct((_B, _TOTAL), jnp.float32),
            jax.ShapeDtypeStruct((_B, _NUM_PROPOSALS), jnp.int32),
            jax.ShapeDtypeStruct((_B, _NUM_PROPOSALS), jnp.float32),
        ),
    )(maps, idx48, scr48)


def _sc_nms_scale(r, side, nsel, base, s_ref, mask_ref, idxv_ref, scrv_ref):
    iota = lax.broadcasted_iota(jnp.int32, (16,), 0)

    def init_row(i, _):
        for h in range(2):
            mask_ref[pl.ds(i * 32 + h * 16, 16)] = jnp.where(
                h * 16 + iota < side, 0.0, _NEG_INF)
        return 0

    lax.fori_loop(0, side, init_row, 0)

    out_idx = jnp.zeros((16,), jnp.int32)
    out_scr = jnp.zeros((16,), jnp.float32)
    for k in range(nsel):
        def max_row(i, vmax):
            for h in range(2):
                d = pl.ds(i * 32 + h * 16, 16)
                vmax = jnp.maximum(vmax, s_ref[d] + mask_ref[d])
            return vmax

        m = _lane_reduce(
            lax.fori_loop(0, side, max_row,
                          jnp.full((16,), _NEG_INF, jnp.float32)),
            jnp.maximum)

        def arg_row(i, vmin):
            for h in range(2):
                d = pl.ds(i * 32 + h * 16, 16)
                p = i * 32 + h * 16 + iota
                cand = jnp.where(s_ref[d] + mask_ref[d] == m, p,
                                 jnp.int32(2**30))
                vmin = jnp.minimum(vmin, cand)
            return vmin

        p32 = _lane_reduce(
            lax.fori_loop(0, side, arg_row,
                          jnp.full((16,), 2**30, jnp.int32)),
            jnp.minimum)
        i0 = lax.shift_right_logical(p32, 5)
        j0 = lax.bitwise_and(p32, 31)

        def supp_row(ii, _):
            u = r - jnp.abs(ii - i0)
            for h in range(2):
                pj = h * 16 + iota
                v = jnp.maximum(0, r - jnp.abs(pj - j0))
                cond = 5 * u * v > 2 * r * r
                d = pl.ds(ii * 32 + h * 16, 16)
                mask_ref[d] = jnp.where(cond, _NEG_INF, mask_ref[d])
            return 0

        lax.fori_loop(jnp.maximum(0, i0 - (r - 1)),
                      jnp.minimum(side, i0 + r), supp_row, 0)

        gidx = i0 * side + j0 + base
        out_idx = jnp.where(iota == k, gidx, out_idx)
        out_scr = jnp.where(iota == k, m, out_scr)

    idxv_ref[...] = out_idx
    scrv_ref[...] = out_scr


def _sc_nms_kernel():
    info = plsc.get_sparse_core_info()
    nc = info.num_cores

    @functools.partial(
        pl.kernel,
        mesh=plsc.VectorSubcoreMesh(core_axis_name="c", subcore_axis_name="s"),
        out_type=(
            jax.ShapeDtypeStruct((_B, 48), jnp.int32),
            jax.ShapeDtypeStruct((_B, 48), jnp.float32),
        ),
        scratch_types=[
            pltpu.VMEM((_HW,), jnp.float32),
            pltpu.VMEM((_HW,), jnp.float32),
            pltpu.VMEM((16,), jnp.int32),
            pltpu.VMEM((16,), jnp.float32),
        ],
    )
    def nms(scores_hbm, idx_hbm, scr_hbm, s_v, mask_v, idxv, scrv):
        wid = lax.axis_index("s") * nc + lax.axis_index("c")
        b = wid % _B
        j = wid // _B

        @pl.when(wid < _B * 3)
        def _():
            pltpu.sync_copy(scores_hbm.at[b, j], s_v)
            for jj, (r, side, nsel, base) in enumerate(_SCALES):
                @pl.when(j == jj)
                def _():
                    _sc_nms_scale(r, side, nsel, base, s_v, mask_v, idxv, scrv)
            pltpu.sync_copy(idxv, idx_hbm.at[b, pl.ds(16 * j, 16)])
            pltpu.sync_copy(scrv, scr_hbm.at[b, pl.ds(16 * j, 16)])

    return nms


@jax.jit
def _run(input_tensor):
    packed = _tc_scores(input_tensor.reshape(_B, _C, _HW))
    idx48, scr48 = _sc_nms_kernel()(packed)
    window_scores, proposal_indices, proposal_scores = _assemble(
        packed, idx48, scr48)
    return proposal_indices, proposal_scores, window_scores


def kernel(input_tensor, coordinates_cat, num_proposals, pooling_ratios,
           window_nums_sum, N_list, iou_thresholds):
    return _run(input_tensor)


# speedup vs baseline: 2.4660x; 1.0409x over previous
import functools

import jax
import jax.numpy as jnp
from jax import lax
from jax.experimental import pallas as pl
from jax.experimental.pallas import tpu as pltpu
from jax.experimental.pallas import tpu_sc as plsc

_B, _C, _H, _W = 8, 768, 32, 32
_HW = _H * _W
_SCALES = (
    (4, 29, 6, 0),
    (6, 27, 5, 841),
    (8, 25, 4, 1570),
)
_TOTAL = 2195
_NUM_PROPOSALS = 15
_NEG_INF = float("-inf")


def _lane_reduce(vec, op):
    vals = [vec[i] for i in range(16)]
    while len(vals) > 1:
        vals = [op(vals[i], vals[i + 1]) for i in range(0, len(vals), 2)]
    return vals[0]


def _pool_1d(fm, r):
    def widen(a, w, b):
        return a + jnp.roll(b, -w, axis=1)

    h2 = widen(fm, 1, fm)
    h4 = widen(h2, 2, h2)
    if r == 4:
        hs = h4
    elif r == 6:
        hs = widen(h4, 4, h2)
    else:
        hs = widen(h4, 4, h4)
    v2 = widen(hs, 32, hs)
    v4 = widen(v2, 64, v2)
    if r == 4:
        ps = v4
    elif r == 6:
        ps = widen(v4, 128, v2)
    else:
        ps = widen(v4, 128, v4)
    return ps * (1.0 / float(r * r))


def _tc_body(x_ref, out_ref):
    fm = jnp.sum(x_ref[0], axis=0, keepdims=True)
    for j, (r, _, _, _) in enumerate(_SCALES):
        out_ref[0, j] = _pool_1d(fm, r)[0]


@jax.jit
def _tc_scores(x):
    return pl.pallas_call(
        _tc_body,
        grid=(_B,),
        in_specs=[pl.BlockSpec((1, _C, _HW), lambda b: (b, 0, 0))],
        out_specs=pl.BlockSpec((1, 3, _HW), lambda b: (b, 0, 0)),
        out_shape=jax.ShapeDtypeStruct((_B, 3, _HW), jnp.float32),
        compiler_params=pltpu.CompilerParams(
            dimension_semantics=("parallel",)),
    )(x)


def _asm_body(maps_ref, idx_ref, scr_ref, ws_ref, pidx_ref, pscr_ref):
    parts = []
    for j, (_, side, _, _) in enumerate(_SCALES):
        for i in range(side):
            parts.append(maps_ref[:, j, 32 * i:32 * i + side])
    ws_ref[...] = jnp.concatenate(parts, axis=1)
    ip, sp = [], []
    for j, (_, _, nsel, _) in enumerate(_SCALES):
        ip.append(idx_ref[:, 16 * j:16 * j + nsel])
        sp.append(scr_ref[:, 16 * j:16 * j + nsel])
    pidx_ref[...] = jnp.concatenate(ip, axis=1)
    pscr_ref[...] = jnp.concatenate(sp, axis=1)


def _assemble(maps, idx48, scr48):
    return pl.pallas_call(
        _asm_body,
        out_shape=(
            jax.ShapeDtypeStruct((_B, _TOTAL), jnp.float32),
            jax.ShapeDtypeStruct((_B, _NUM_PROPOSALS), jnp.int32),
            jax.ShapeDtypeStruct((_B, _NUM_PROPOSALS), jnp.float32),
        ),
    )(maps, idx48, scr48)


def _sc_nms_scale(r, side, nsel, base, s_ref, mask_ref, idxv_ref, scrv_ref):
    iota = lax.broadcasted_iota(jnp.int32, (16,), 0)

    def init_row(i, _):
        for h in range(2):
            mask_ref[pl.ds(i * 32 + h * 16, 16)] = jnp.where(
                h * 16 + iota < side, 0.0, _NEG_INF)
        return 0

    lax.fori_loop(0, side, init_row, 0)

    def pick(k, carry):
        out_idx, out_scr = carry

        def max_row(i, vmax):
            for h in range(2):
                d = pl.ds(i * 32 + h * 16, 16)
                vmax = jnp.maximum(vmax, s_ref[d] + mask_ref[d])
            return vmax

        m = _lane_reduce(
            lax.fori_loop(0, side, max_row,
                          jnp.full((16,), _NEG_INF, jnp.float32)),
            jnp.maximum)

        def arg_row(i, vmin):
            for h in range(2):
                d = pl.ds(i * 32 + h * 16, 16)
                p = i * 32 + h * 16 + iota
                cand = jnp.where(s_ref[d] + mask_ref[d] == m, p,
                                 jnp.int32(2**30))
                vmin = jnp.minimum(vmin, cand)
            return vmin

        p32 = _lane_reduce(
            lax.fori_loop(0, side, arg_row,
                          jnp.full((16,), 2**30, jnp.int32)),
            jnp.minimum)
        i0 = lax.shift_right_logical(p32, 5)
        j0 = lax.bitwise_and(p32, 31)

        def supp_row(ii, _):
            u = r - jnp.abs(ii - i0)
            for h in range(2):
                pj = h * 16 + iota
                v = jnp.maximum(0, r - jnp.abs(pj - j0))
                cond = 5 * u * v > 2 * r * r
                d = pl.ds(ii * 32 + h * 16, 16)
                mask_ref[d] = jnp.where(cond, _NEG_INF, mask_ref[d])
            return 0

        lax.fori_loop(jnp.maximum(0, i0 - (r - 1)),
                      jnp.minimum(side, i0 + r), supp_row, 0)

        gidx = i0 * side + j0 + base
        out_idx = jnp.where(iota == k, gidx, out_idx)
        out_scr = jnp.where(iota == k, m, out_scr)
        return out_idx, out_scr

    out_idx, out_scr = lax.fori_loop(
        0, nsel, pick,
        (jnp.zeros((16,), jnp.int32), jnp.zeros((16,), jnp.float32)))
    idxv_ref[...] = out_idx
    scrv_ref[...] = out_scr


def _sc_nms_kernel():
    info = plsc.get_sparse_core_info()
    nc = info.num_cores

    @functools.partial(
        pl.kernel,
        mesh=plsc.VectorSubcoreMesh(core_axis_name="c", subcore_axis_name="s"),
        out_type=(
            jax.ShapeDtypeStruct((_B, 48), jnp.int32),
            jax.ShapeDtypeStruct((_B, 48), jnp.float32),
        ),
        scratch_types=[
            pltpu.VMEM((_HW,), jnp.float32),
            pltpu.VMEM((_HW,), jnp.float32),
            pltpu.VMEM((16,), jnp.int32),
            pltpu.VMEM((16,), jnp.float32),
        ],
    )
    def nms(scores_hbm, idx_hbm, scr_hbm, s_v, mask_v, idxv, scrv):
        wid = lax.axis_index("s") * nc + lax.axis_index("c")
        b = wid % _B
        j = wid // _B

        @pl.when(wid < _B * 3)
        def _():
            pltpu.sync_copy(scores_hbm.at[b, j], s_v)
            r = jnp.where(j == 0, 4, jnp.where(j == 1, 6, 8))
            side = 33 - r
            nsel = jnp.where(j == 0, 6, jnp.where(j == 1, 5, 4))
            base = jnp.where(j == 0, 0, jnp.where(j == 1, 841, 1570))
            _sc_nms_scale(r, side, nsel, base, s_v, mask_v, idxv, scrv)
            pltpu.sync_copy(idxv, idx_hbm.at[b, pl.ds(16 * j, 16)])
            pltpu.sync_copy(scrv, scr_hbm.at[b, pl.ds(16 * j, 16)])

    return nms


@jax.jit
def _run(input_tensor):
    packed = _tc_scores(input_tensor.reshape(_B, _C, _HW))
    idx48, scr48 = _sc_nms_kernel()(packed)
    window_scores, proposal_indices, proposal_scores = _assemble(
        packed, idx48, scr48)
    return proposal_indices, proposal_scores, window_scores


def kernel(input_tensor, coordinates_cat, num_proposals, pooling_ratios,
           window_nums_sum, N_list, iou_thresholds):
    return _run(input_tensor)


# speedup vs baseline: 2.4834x; 1.0071x over previous
import functools

import jax
import jax.numpy as jnp
from jax import lax
from jax.experimental import pallas as pl
from jax.experimental.pallas import tpu as pltpu
from jax.experimental.pallas import tpu_sc as plsc

_B, _C, _H, _W = 8, 768, 32, 32
_HW = _H * _W
_SCALES = (
    (4, 29, 6, 0),
    (6, 27, 5, 841),
    (8, 25, 4, 1570),
)
_TOTAL = 2195
_NUM_PROPOSALS = 15
_NEG_INF = float("-inf")


def _lane_reduce(vec, op):
    vals = [vec[i] for i in range(16)]
    while len(vals) > 1:
        vals = [op(vals[i], vals[i + 1]) for i in range(0, len(vals), 2)]
    return vals[0]


def _pool_1d(fm, r):
    def widen(a, w, b):
        return a + jnp.roll(b, -w, axis=1)

    h2 = widen(fm, 1, fm)
    h4 = widen(h2, 2, h2)
    if r == 4:
        hs = h4
    elif r == 6:
        hs = widen(h4, 4, h2)
    else:
        hs = widen(h4, 4, h4)
    v2 = widen(hs, 32, hs)
    v4 = widen(v2, 64, v2)
    if r == 4:
        ps = v4
    elif r == 6:
        ps = widen(v4, 128, v2)
    else:
        ps = widen(v4, 128, v4)
    return ps * (1.0 / float(r * r))


def _tc_body(xa_ref, xb_ref, out_ref):
    fm = (jnp.sum(xa_ref[0], axis=0, keepdims=True)
          + jnp.sum(xb_ref[0], axis=0, keepdims=True))
    for j, (r, _, _, _) in enumerate(_SCALES):
        out_ref[0, j] = _pool_1d(fm, r)[0]


@jax.jit
def _tc_scores(x):
    half = _C // 2
    return pl.pallas_call(
        _tc_body,
        grid=(_B,),
        in_specs=[
            pl.BlockSpec((1, half, _HW), lambda b: (b, 0, 0)),
            pl.BlockSpec((1, half, _HW), lambda b: (b, 1, 0)),
        ],
        out_specs=pl.BlockSpec((1, 3, _HW), lambda b: (b, 0, 0)),
        out_shape=jax.ShapeDtypeStruct((_B, 3, _HW), jnp.float32),
        compiler_params=pltpu.CompilerParams(
            dimension_semantics=("parallel",)),
    )(x, x)


def _asm_body(maps_ref, idx_ref, scr_ref, ws_ref, pidx_ref, pscr_ref):
    parts = []
    for j, (_, side, _, _) in enumerate(_SCALES):
        for i in range(side):
            parts.append(maps_ref[:, j, 32 * i:32 * i + side])
    ws_ref[...] = jnp.concatenate(parts, axis=1)
    ip, sp = [], []
    for j, (_, _, nsel, _) in enumerate(_SCALES):
        ip.append(idx_ref[:, 16 * j:16 * j + nsel])
        sp.append(scr_ref[:, 16 * j:16 * j + nsel])
    pidx_ref[...] = jnp.concatenate(ip, axis=1)
    pscr_ref[...] = jnp.concatenate(sp, axis=1)


def _assemble(maps, idx48, scr48):
    return pl.pallas_call(
        _asm_body,
        out_shape=(
            jax.ShapeDtypeStruct((_B, _TOTAL), jnp.float32),
            jax.ShapeDtypeStruct((_B, _NUM_PROPOSALS), jnp.int32),
            jax.ShapeDtypeStruct((_B, _NUM_PROPOSALS), jnp.float32),
        ),
    )(maps, idx48, scr48)


def _sc_nms_scale(r, side, nsel, base, s_ref, mask_ref, idxv_ref, scrv_ref):
    iota = lax.broadcasted_iota(jnp.int32, (16,), 0)

    def init_row(i, _):
        for h in range(2):
            mask_ref[pl.ds(i * 32 + h * 16, 16)] = jnp.where(
                h * 16 + iota < side, 0.0, _NEG_INF)
        return 0

    lax.fori_loop(0, side, init_row, 0)

    def pick(k, carry):
        out_idx, out_scr = carry

        def max_row(i, vmax):
            for h in range(2):
                d = pl.ds(i * 32 + h * 16, 16)
                vmax = jnp.maximum(vmax, s_ref[d] + mask_ref[d])
            return vmax

        m = _lane_reduce(
            lax.fori_loop(0, side, max_row,
                          jnp.full((16,), _NEG_INF, jnp.float32)),
            jnp.maximum)

        def arg_row(i, vmin):
            for h in range(2):
                d = pl.ds(i * 32 + h * 16, 16)
                p = i * 32 + h * 16 + iota
                cand = jnp.where(s_ref[d] + mask_ref[d] == m, p,
                                 jnp.int32(2**30))
                vmin = jnp.minimum(vmin, cand)
            return vmin

        p32 = _lane_reduce(
            lax.fori_loop(0, side, arg_row,
                          jnp.full((16,), 2**30, jnp.int32)),
            jnp.minimum)
        i0 = lax.shift_right_logical(p32, 5)
        j0 = lax.bitwise_and(p32, 31)

        def supp_row(ii, _):
            u = r - jnp.abs(ii - i0)
            for h in range(2):
                pj = h * 16 + iota
                v = jnp.maximum(0, r - jnp.abs(pj - j0))
                cond = 5 * u * v > 2 * r * r
                d = pl.ds(ii * 32 + h * 16, 16)
                mask_ref[d] = jnp.where(cond, _NEG_INF, mask_ref[d])
            return 0

        lax.fori_loop(jnp.maximum(0, i0 - (r - 1)),
                      jnp.minimum(side, i0 + r), supp_row, 0)

        gidx = i0 * side + j0 + base
        out_idx = jnp.where(iota == k, gidx, out_idx)
        out_scr = jnp.where(iota == k, m, out_scr)
        return out_idx, out_scr

    out_idx, out_scr = lax.fori_loop(
        0, nsel, pick,
        (jnp.zeros((16,), jnp.int32), jnp.zeros((16,), jnp.float32)))
    idxv_ref[...] = out_idx
    scrv_ref[...] = out_scr


def _sc_nms_kernel():
    info = plsc.get_sparse_core_info()
    nc = info.num_cores

    @functools.partial(
        pl.kernel,
        mesh=plsc.VectorSubcoreMesh(core_axis_name="c", subcore_axis_name="s"),
        out_type=(
            jax.ShapeDtypeStruct((_B, 48), jnp.int32),
            jax.ShapeDtypeStruct((_B, 48), jnp.float32),
        ),
        scratch_types=[
            pltpu.VMEM((_HW,), jnp.float32),
            pltpu.VMEM((_HW,), jnp.float32),
            pltpu.VMEM((16,), jnp.int32),
            pltpu.VMEM((16,), jnp.float32),
        ],
    )
    def nms(scores_hbm, idx_hbm, scr_hbm, s_v, mask_v, idxv, scrv):
        wid = lax.axis_index("s") * nc + lax.axis_index("c")
        b = wid % _B
        j = wid // _B

        @pl.when(wid < _B * 3)
        def _():
            pltpu.sync_copy(scores_hbm.at[b, j], s_v)
            r = jnp.where(j == 0, 4, jnp.where(j == 1, 6, 8))
            side = 33 - r
            nsel = jnp.where(j == 0, 6, jnp.where(j == 1, 5, 4))
            base = jnp.where(j == 0, 0, jnp.where(j == 1, 841, 1570))
            _sc_nms_scale(r, side, nsel, base, s_v, mask_v, idxv, scrv)
            pltpu.sync_copy(idxv, idx_hbm.at[b, pl.ds(16 * j, 16)])
            pltpu.sync_copy(scrv, scr_hbm.at[b, pl.ds(16 * j, 16)])

    return nms


@jax.jit
def _run(input_tensor):
    packed = _tc_scores(input_tensor.reshape(_B, _C, _HW))
    idx48, scr48 = _sc_nms_kernel()(packed)
    window_scores, proposal_indices, proposal_scores = _assemble(
        packed, idx48, scr48)
    return proposal_indices, proposal_scores, window_scores


def kernel(input_tensor, coordinates_cat, num_proposals, pooling_ratios,
           window_nums_sum, N_list, iou_thresholds):
    return _run(input_tensor)
